# baseline (speedup 1.0000x reference)
import functools

import jax
import jax.numpy as jnp
from jax import lax
from jax.experimental import pallas as pl
from jax.experimental.pallas import tpu as pltpu
from jax.experimental.pallas import tpu_sc as plsc

_B, _T, _C, _H, _M, _K = 1, 2048, 1024, 16, 8192, 3
_DH = _C // _H
_BT = 256
_BM = 1024


def _qkv_body(x_ref, w_ref, b_ref, oqh_ref, okt_ref, ovh_ref, oqt_ref):
    val = (
        jnp.dot(x_ref[...], w_ref[...], preferred_element_type=jnp.float32)
        + b_ref[...][None, :]
    )
    for h in range(_H):
        oqh_ref[h] = val[:, h * _DH:(h + 1) * _DH]
        okt_ref[h] = val[:, _C + h * _DH:_C + (h + 1) * _DH].T
        ovh_ref[h] = val[:, 2 * _C + h * _DH:2 * _C + (h + 1) * _DH]
    oqt_ref[...] = val[:, :_C].T


def _qkv(x, w, b):
    return pl.pallas_call(
        _qkv_body,
        grid=(_T // _BT,),
        in_specs=[
            pl.BlockSpec((_BT, _C), lambda i: (i, 0)),
            pl.BlockSpec((_C, 3 * _C), lambda i: (0, 0)),
            pl.BlockSpec((3 * _C,), lambda i: (0,)),
        ],
        out_specs=[
            pl.BlockSpec((_H, _BT, _DH), lambda i: (0, i, 0)),
            pl.BlockSpec((_H, _DH, _BT), lambda i: (0, 0, i)),
            pl.BlockSpec((_H, _BT, _DH), lambda i: (0, i, 0)),
            pl.BlockSpec((_C, _BT), lambda i: (0, i)),
        ],
        out_shape=[
            jax.ShapeDtypeStruct((_H, _T, _DH), jnp.float32),
            jax.ShapeDtypeStruct((_H, _DH, _T), jnp.float32),
            jax.ShapeDtypeStruct((_H, _T, _DH), jnp.float32),
            jax.ShapeDtypeStruct((_C, _T), jnp.float32),
        ],
    )(x, w, b)


_NSEG = 4
_WSEG = _T // _NSEG
_BTA = 512
_BTC = 512
_SPB = _BTA // _WSEG


def _attn_body(q_ref, kt_ref, v_ref, o_ref, l_sc, a_sc):
    i = pl.program_id(1)
    f32 = jnp.float32
    scale = 1.0 / jnp.sqrt(f32(_DH))
    l_sc[...] = jnp.zeros((_BTA, 8), f32)
    a_sc[...] = jnp.zeros((_BTA, 2 * _DH), f32)

    def seg(lo, mask_d):
        for hh in range(2):
            q = q_ref[hh] * scale
            s = jnp.dot(q, kt_ref[hh, :, lo:lo + _WSEG],
                        preferred_element_type=f32)
            if mask_d is None:
                p = jnp.exp(s)
            else:
                rows = lax.broadcasted_iota(jnp.int32, (_BTA, _WSEG), 0)
                cols = (mask_d * _WSEG
                        + lax.broadcasted_iota(jnp.int32, (_BTA, _WSEG), 1))
                p = jnp.where(cols <= rows, jnp.exp(s), f32(0.0))
            l_sc[:, hh:hh + 1] += jnp.sum(p, axis=1, keepdims=True)
            a_sc[:, hh * _DH:(hh + 1) * _DH] += jnp.dot(
                p, v_ref[hh, lo:lo + _WSEG, :], preferred_element_type=f32)

    for c in range(_NSEG):
        lo = c * _WSEG

        @pl.when(c < i * _SPB)
        def _(lo=lo):
            seg(lo, None)

        for d in range(_SPB):
            @pl.when(c == i * _SPB + d)
            def _(lo=lo, d=d):
                seg(lo, d)

    o_ref[0] = a_sc[:, :_DH] / l_sc[:, 0:1]
    o_ref[1] = a_sc[:, _DH:] / l_sc[:, 1:2]


def _attn(qh, kth, vh):
    return pl.pallas_call(
        _attn_body,
        grid=(_H // 2, _T // _BTA),
        in_specs=[
            pl.BlockSpec((2, _BTA, _DH), lambda h, i: (h, i, 0)),
            pl.BlockSpec((2, _DH, _T), lambda h, i: (h, 0, 0)),
            pl.BlockSpec((2, _T, _DH), lambda h, i: (h, 0, 0)),
        ],
        out_specs=pl.BlockSpec((2, _BTA, _DH), lambda h, i: (h, i, 0)),
        out_shape=jax.ShapeDtypeStruct((_H, _T, _DH), jnp.float32),
        scratch_shapes=[pltpu.VMEM((_BTA, 8), jnp.float32),
                        pltpu.VMEM((_BTA, 2 * _DH), jnp.float32)],
    )(qh, kth, vh)


def _topk_body(qt_ref, kb_ref, o_ref, rv_ref, ri_ref):
    f32, i32 = jnp.float32, jnp.int32
    j = pl.program_id(0)
    i = pl.program_id(1)

    @pl.when(j == 0)
    def _():
        rv_ref[i] = jnp.full((8, _BT), jnp.inf, f32)
        ri_ref[i] = jnp.full((8, _BT), 2 ** 30, i32)

    kb = kb_ref[...]
    kn = jnp.sum(kb * kb, axis=1, keepdims=True)
    qk = jnp.dot(kb, qt_ref[...], preferred_element_type=f32)
    s = kn - 2.0 * qk

    rows = j * _BM + lax.broadcasted_iota(i32, (_BM, _BT), 0)
    lv, li = [], []
    for t in range(_K):
        mn = jnp.min(s, axis=0, keepdims=True)
        im = jnp.min(jnp.where(s == mn, rows, 2 ** 30), axis=0, keepdims=True)
        lv.append(mn)
        li.append(im)
        if t < _K - 1:
            s = jnp.where(rows == im, jnp.float32(jnp.inf), s)

    rv = rv_ref[i]
    ri = ri_ref[i]
    padv = jnp.full((1, _BT), jnp.inf, f32)
    padi = jnp.full((1, _BT), 2 ** 30, i32)
    cv = jnp.concatenate(
        [rv[0:1], rv[1:2], rv[2:3]] + lv + [padv, padv], axis=0)
    ci = jnp.concatenate(
        [ri[0:1], ri[1:2], ri[2:3]] + li + [padi, padi], axis=0)
    pos = lax.broadcasted_iota(i32, (8, _BT), 0)
    nv, ni = [], []
    for _t in range(_K):
        mn = jnp.min(cv, axis=0, keepdims=True)
        p = jnp.min(jnp.where(cv == mn, pos, 8), axis=0, keepdims=True)
        nv.append(mn)
        ni.append(jnp.min(jnp.where(pos == p, ci, 2 ** 30), axis=0,
                          keepdims=True))
        cv = jnp.where(pos == p, jnp.float32(jnp.inf), cv)
    rv_ref[i] = jnp.concatenate(nv + [padv] * 5, axis=0)
    ri_ref[i] = jnp.concatenate(ni + [padi] * 5, axis=0)

    @pl.when(j == _M // _BM - 1)
    def _():
        o_ref[...] = jnp.concatenate(ni, axis=0)


def _topk(qt, mem_flat):
    return pl.pallas_call(
        _topk_body,
        grid=(_M // _BM, _T // _BT),
        in_specs=[
            pl.BlockSpec((_C, _BT), lambda j, i: (0, i)),
            pl.BlockSpec((_BM, _C), lambda j, i: (j, 0)),
        ],
        out_specs=pl.BlockSpec((_K, _BT), lambda j, i: (0, i)),
        out_shape=jax.ShapeDtypeStruct((_K, _T), jnp.int32),
        scratch_shapes=[pltpu.VMEM((_T // _BT, 8, _BT), jnp.float32),
                        pltpu.VMEM((_T // _BT, 8, _BT), jnp.int32)],
    )(qt, mem_flat)


def _gather_rows(table, idx):
    info = plsc.get_sparse_core_info()
    nw = info.num_cores * info.num_subcores
    nrows = idx.shape[0]
    bpw = nrows // nw
    chunk = 24
    width = table.shape[1]
    mesh = plsc.VectorSubcoreMesh(core_axis_name="c", subcore_axis_name="s")

    nch = bpw // chunk

    @functools.partial(
        pl.kernel,
        mesh=mesh,
        out_type=jax.ShapeDtypeStruct((nrows, width), jnp.float32),
        scratch_types=[
            pltpu.VMEM((chunk,), jnp.int32),
            pltpu.VMEM((chunk, width), jnp.float32),
            pltpu.VMEM((chunk,), jnp.int32),
            pltpu.VMEM((chunk, width), jnp.float32),
            pltpu.SemaphoreType.DMA,
            pltpu.SemaphoreType.DMA,
        ],
    )
    def gk(table_hbm, idx_hbm, out_hbm, idx_v0, rows_v0, idx_v1, rows_v1,
           sem0, sem1):
        wid = lax.axis_index("s") * info.num_cores + lax.axis_index("c")
        base = wid * bpw
        bufs = [(idx_v0, rows_v0, sem0), (idx_v1, rows_v1, sem1)]

        pltpu.sync_copy(idx_hbm.at[pl.ds(base, chunk)], idx_v0)
        handles = [None] * nch
        handles[0] = pltpu.async_copy(table_hbm.at[idx_v0], rows_v0, sem0)
        for c in range(nch):
            _, rv, _ = bufs[c % 2]
            if c + 1 < nch:
                niv, nrv, nsm = bufs[(c + 1) % 2]
                off = base + (c + 1) * chunk
                pltpu.sync_copy(idx_hbm.at[pl.ds(off, chunk)], niv)
                handles[c + 1] = pltpu.async_copy(table_hbm.at[niv], nrv, nsm)
            handles[c].wait()
            pltpu.sync_copy(rv, out_hbm.at[pl.ds(base + c * chunk, chunk)])

    return gk(table, idx)


def _combine_body(q_ref, y_ref, kv_ref, g_ref, w_ref, b_ref, o_ref):
    f32 = jnp.float32
    ch = lax.broadcasted_iota(jnp.int32, (_C, _H), 0) // _DH
    hh = lax.broadcasted_iota(jnp.int32, (_C, _H), 1)
    seg = (ch == hh).astype(f32)
    ch2 = lax.broadcasted_iota(jnp.int32, (_H, _C), 1) // _DH
    hh2 = lax.broadcasted_iota(jnp.int32, (_H, _C), 0)
    seg_t = (ch2 == hh2).astype(f32)
    q = jnp.concatenate([q_ref[h] for h in range(_H)], axis=1)
    logits = []
    for kk in range(_K):
        mk = kv_ref[kk, :, :_C]
        logits.append(
            jnp.dot(q * mk, seg, preferred_element_type=f32) * 0.125
        )
    m = jnp.maximum(jnp.maximum(logits[0], logits[1]), logits[2])
    es = [jnp.exp(l - m) for l in logits]
    den = es[0] + es[1] + es[2]
    mem = jnp.zeros((_BTC, _C), f32)
    for kk in range(_K):
        w_full = jnp.dot(es[kk] / den, seg_t,
                         preferred_element_type=f32)
        mem = mem + w_full * kv_ref[kk, :, _C:]
    g = g_ref[...][None, :]
    y = jnp.concatenate([y_ref[h] for h in range(_H)], axis=1)
    comb = mem * g + y * (1.0 - g)
    o_ref[...] = (
        jnp.dot(comb, w_ref[...], preferred_element_type=f32)
        + b_ref[...][None, :]
    )


def _combine(qh, yh, kv3, gfull, wp, bp):
    return pl.pallas_call(
        _combine_body,
        grid=(_T // _BTC,),
        in_specs=[
            pl.BlockSpec((_H, _BTC, _DH), lambda i: (0, i, 0)),
            pl.BlockSpec((_H, _BTC, _DH), lambda i: (0, i, 0)),
            pl.BlockSpec((_K, _BTC, 2 * _C), lambda i: (0, i, 0)),
            pl.BlockSpec((_C,), lambda i: (0,)),
            pl.BlockSpec((_C, _C), lambda i: (0, 0)),
            pl.BlockSpec((_C,), lambda i: (0,)),
        ],
        out_specs=pl.BlockSpec((_BTC, _C), lambda i: (i, 0)),
        out_shape=jax.ShapeDtypeStruct((_T, _C), jnp.float32),
    )(qh, yh, kv3, gfull, wp, bp)


def kernel(x, memory_db, W_attn, b_attn, W_proj, b_proj, gate_bias):
    x2 = x.reshape(_T, _C)
    mem_flat = memory_db.reshape(_M, 2 * _C)
    qh, kth, vh, qt = _qkv(x2, W_attn, b_attn)
    idx = _topk(qt, mem_flat)
    kvs = _gather_rows(mem_flat, idx.reshape(_K * _T))
    kv3 = kvs.reshape(_K, _T, 2 * _C)
    yh = _attn(qh, kth, vh)
    gfull = jnp.repeat(gate_bias.reshape(_H), _DH)
    out = _combine(qh, yh, kv3, gfull, W_proj, b_proj)
    return out.reshape(_B, _T, _C)


# speedup vs baseline: 1.0105x; 1.0105x over previous
import functools

import jax
import jax.numpy as jnp
from jax import lax
from jax.experimental import pallas as pl
from jax.experimental.pallas import tpu as pltpu
from jax.experimental.pallas import tpu_sc as plsc

_B, _T, _C, _H, _M, _K = 1, 2048, 1024, 16, 8192, 3
_DH = _C // _H
_BT = 256
_BM = 2048


def _qkv_body(x_ref, w_ref, b_ref, oqh_ref, okt_ref, ovh_ref, oqt_ref):
    val = (
        jnp.dot(x_ref[...], w_ref[...], preferred_element_type=jnp.float32)
        + b_ref[...][None, :]
    )
    for h in range(_H):
        oqh_ref[h] = val[:, h * _DH:(h + 1) * _DH]
        okt_ref[h] = val[:, _C + h * _DH:_C + (h + 1) * _DH].T
        ovh_ref[h] = val[:, 2 * _C + h * _DH:2 * _C + (h + 1) * _DH]
    oqt_ref[...] = val[:, :_C].T


def _qkv(x, w, b):
    return pl.pallas_call(
        _qkv_body,
        grid=(_T // _BT,),
        in_specs=[
            pl.BlockSpec((_BT, _C), lambda i: (i, 0)),
            pl.BlockSpec((_C, 3 * _C), lambda i: (0, 0)),
            pl.BlockSpec((3 * _C,), lambda i: (0,)),
        ],
        out_specs=[
            pl.BlockSpec((_H, _BT, _DH), lambda i: (0, i, 0)),
            pl.BlockSpec((_H, _DH, _BT), lambda i: (0, 0, i)),
            pl.BlockSpec((_H, _BT, _DH), lambda i: (0, i, 0)),
            pl.BlockSpec((_C, _BT), lambda i: (0, i)),
        ],
        out_shape=[
            jax.ShapeDtypeStruct((_H, _T, _DH), jnp.float32),
            jax.ShapeDtypeStruct((_H, _DH, _T), jnp.float32),
            jax.ShapeDtypeStruct((_H, _T, _DH), jnp.float32),
            jax.ShapeDtypeStruct((_C, _T), jnp.float32),
        ],
    )(x, w, b)


_NSEG = 4
_WSEG = _T // _NSEG
_BTA = 512
_BTC = 512
_SPB = _BTA // _WSEG


def _attn_body(q_ref, kt_ref, v_ref, o_ref, l_sc, a_sc):
    i = pl.program_id(1)
    f32 = jnp.float32
    scale = 1.0 / jnp.sqrt(f32(_DH))
    l_sc[...] = jnp.zeros((_BTA, 8), f32)
    a_sc[...] = jnp.zeros((_BTA, 2 * _DH), f32)

    def seg(lo, mask_d):
        for hh in range(2):
            q = q_ref[hh] * scale
            s = jnp.dot(q, kt_ref[hh, :, lo:lo + _WSEG],
                        preferred_element_type=f32)
            if mask_d is None:
                p = jnp.exp(s)
            else:
                rows = lax.broadcasted_iota(jnp.int32, (_BTA, _WSEG), 0)
                cols = (mask_d * _WSEG
                        + lax.broadcasted_iota(jnp.int32, (_BTA, _WSEG), 1))
                p = jnp.where(cols <= rows, jnp.exp(s), f32(0.0))
            l_sc[:, hh:hh + 1] += jnp.sum(p, axis=1, keepdims=True)
            a_sc[:, hh * _DH:(hh + 1) * _DH] += jnp.dot(
                p, v_ref[hh, lo:lo + _WSEG, :], preferred_element_type=f32)

    for c in range(_NSEG):
        lo = c * _WSEG

        @pl.when(c < i * _SPB)
        def _(lo=lo):
            seg(lo, None)

        for d in range(_SPB):
            @pl.when(c == i * _SPB + d)
            def _(lo=lo, d=d):
                seg(lo, d)

    o_ref[0] = a_sc[:, :_DH] / l_sc[:, 0:1]
    o_ref[1] = a_sc[:, _DH:] / l_sc[:, 1:2]


def _attn(qh, kth, vh):
    return pl.pallas_call(
        _attn_body,
        grid=(_H // 2, _T // _BTA),
        in_specs=[
            pl.BlockSpec((2, _BTA, _DH), lambda h, i: (h, i, 0)),
            pl.BlockSpec((2, _DH, _T), lambda h, i: (h, 0, 0)),
            pl.BlockSpec((2, _T, _DH), lambda h, i: (h, 0, 0)),
        ],
        out_specs=pl.BlockSpec((2, _BTA, _DH), lambda h, i: (h, i, 0)),
        out_shape=jax.ShapeDtypeStruct((_H, _T, _DH), jnp.float32),
        scratch_shapes=[pltpu.VMEM((_BTA, 8), jnp.float32),
                        pltpu.VMEM((_BTA, 2 * _DH), jnp.float32)],
    )(qh, kth, vh)


def _topk_body(qt_ref, kb_ref, o_ref, rv_ref, ri_ref):
    f32, i32 = jnp.float32, jnp.int32
    j = pl.program_id(0)
    i = pl.program_id(1)

    @pl.when(j == 0)
    def _():
        rv_ref[i] = jnp.full((8, _BT), jnp.inf, f32)
        ri_ref[i] = jnp.full((8, _BT), 2 ** 30, i32)

    kb = kb_ref[...]
    kn = jnp.sum(kb * kb, axis=1, keepdims=True)
    qk = jnp.dot(kb, qt_ref[...], preferred_element_type=f32)
    s = kn - 2.0 * qk

    rows = j * _BM + lax.broadcasted_iota(i32, (_BM, _BT), 0)
    lv, li = [], []
    for t in range(_K):
        mn = jnp.min(s, axis=0, keepdims=True)
        im = jnp.min(jnp.where(s == mn, rows, 2 ** 30), axis=0, keepdims=True)
        lv.append(mn)
        li.append(im)
        if t < _K - 1:
            s = jnp.where(rows == im, jnp.float32(jnp.inf), s)

    rv = rv_ref[i]
    ri = ri_ref[i]
    padv = jnp.full((1, _BT), jnp.inf, f32)
    padi = jnp.full((1, _BT), 2 ** 30, i32)
    cv = jnp.concatenate(
        [rv[0:1], rv[1:2], rv[2:3]] + lv + [padv, padv], axis=0)
    ci = jnp.concatenate(
        [ri[0:1], ri[1:2], ri[2:3]] + li + [padi, padi], axis=0)
    pos = lax.broadcasted_iota(i32, (8, _BT), 0)
    nv, ni = [], []
    for _t in range(_K):
        mn = jnp.min(cv, axis=0, keepdims=True)
        p = jnp.min(jnp.where(cv == mn, pos, 8), axis=0, keepdims=True)
        nv.append(mn)
        ni.append(jnp.min(jnp.where(pos == p, ci, 2 ** 30), axis=0,
                          keepdims=True))
        cv = jnp.where(pos == p, jnp.float32(jnp.inf), cv)
    rv_ref[i] = jnp.concatenate(nv + [padv] * 5, axis=0)
    ri_ref[i] = jnp.concatenate(ni + [padi] * 5, axis=0)

    @pl.when(j == _M // _BM - 1)
    def _():
        o_ref[...] = jnp.concatenate(ni, axis=0)


def _topk(qt, mem_flat):
    return pl.pallas_call(
        _topk_body,
        grid=(_M // _BM, _T // _BT),
        in_specs=[
            pl.BlockSpec((_C, _BT), lambda j, i: (0, i)),
            pl.BlockSpec((_BM, _C), lambda j, i: (j, 0)),
        ],
        out_specs=pl.BlockSpec((_K, _BT), lambda j, i: (0, i)),
        out_shape=jax.ShapeDtypeStruct((_K, _T), jnp.int32),
        scratch_shapes=[pltpu.VMEM((_T // _BT, 8, _BT), jnp.float32),
                        pltpu.VMEM((_T // _BT, 8, _BT), jnp.int32)],
    )(qt, mem_flat)


def _gather_rows(table, idx):
    info = plsc.get_sparse_core_info()
    nw = info.num_cores * info.num_subcores
    nrows = idx.shape[0]
    bpw = nrows // nw
    chunk = 24
    width = table.shape[1]
    mesh = plsc.VectorSubcoreMesh(core_axis_name="c", subcore_axis_name="s")

    nch = bpw // chunk

    @functools.partial(
        pl.kernel,
        mesh=mesh,
        out_type=jax.ShapeDtypeStruct((nrows, width), jnp.float32),
        scratch_types=[
            pltpu.VMEM((chunk,), jnp.int32),
            pltpu.VMEM((chunk, width), jnp.float32),
            pltpu.VMEM((chunk,), jnp.int32),
            pltpu.VMEM((chunk, width), jnp.float32),
            pltpu.SemaphoreType.DMA,
            pltpu.SemaphoreType.DMA,
        ],
    )
    def gk(table_hbm, idx_hbm, out_hbm, idx_v0, rows_v0, idx_v1, rows_v1,
           sem0, sem1):
        wid = lax.axis_index("s") * info.num_cores + lax.axis_index("c")
        base = wid * bpw
        bufs = [(idx_v0, rows_v0, sem0), (idx_v1, rows_v1, sem1)]

        pltpu.sync_copy(idx_hbm.at[pl.ds(base, chunk)], idx_v0)
        handles = [None] * nch
        handles[0] = pltpu.async_copy(table_hbm.at[idx_v0], rows_v0, sem0)
        for c in range(nch):
            _, rv, _ = bufs[c % 2]
            if c + 1 < nch:
                niv, nrv, nsm = bufs[(c + 1) % 2]
                off = base + (c + 1) * chunk
                pltpu.sync_copy(idx_hbm.at[pl.ds(off, chunk)], niv)
                handles[c + 1] = pltpu.async_copy(table_hbm.at[niv], nrv, nsm)
            handles[c].wait()
            pltpu.sync_copy(rv, out_hbm.at[pl.ds(base + c * chunk, chunk)])

    return gk(table, idx)


def _combine_body(q_ref, y_ref, kv_ref, g_ref, w_ref, b_ref, o_ref):
    f32 = jnp.float32
    ch = lax.broadcasted_iota(jnp.int32, (_C, _H), 0) // _DH
    hh = lax.broadcasted_iota(jnp.int32, (_C, _H), 1)
    seg = (ch == hh).astype(f32)
    ch2 = lax.broadcasted_iota(jnp.int32, (_H, _C), 1) // _DH
    hh2 = lax.broadcasted_iota(jnp.int32, (_H, _C), 0)
    seg_t = (ch2 == hh2).astype(f32)
    q = jnp.concatenate([q_ref[h] for h in range(_H)], axis=1)
    logits = []
    for kk in range(_K):
        mk = kv_ref[kk, :, :_C]
        logits.append(
            jnp.dot(q * mk, seg, preferred_element_type=f32) * 0.125
        )
    m = jnp.maximum(jnp.maximum(logits[0], logits[1]), logits[2])
    es = [jnp.exp(l - m) for l in logits]
    den = es[0] + es[1] + es[2]
    mem = jnp.zeros((_BTC, _C), f32)
    for kk in range(_K):
        w_full = jnp.dot(es[kk] / den, seg_t,
                         preferred_element_type=f32)
        mem = mem + w_full * kv_ref[kk, :, _C:]
    g = g_ref[...][None, :]
    y = jnp.concatenate([y_ref[h] for h in range(_H)], axis=1)
    comb = mem * g + y * (1.0 - g)
    o_ref[...] = (
        jnp.dot(comb, w_ref[...], preferred_element_type=f32)
        + b_ref[...][None, :]
    )


def _combine(qh, yh, kv3, gfull, wp, bp):
    return pl.pallas_call(
        _combine_body,
        grid=(_T // _BTC,),
        in_specs=[
            pl.BlockSpec((_H, _BTC, _DH), lambda i: (0, i, 0)),
            pl.BlockSpec((_H, _BTC, _DH), lambda i: (0, i, 0)),
            pl.BlockSpec((_K, _BTC, 2 * _C), lambda i: (0, i, 0)),
            pl.BlockSpec((_C,), lambda i: (0,)),
            pl.BlockSpec((_C, _C), lambda i: (0, 0)),
            pl.BlockSpec((_C,), lambda i: (0,)),
        ],
        out_specs=pl.BlockSpec((_BTC, _C), lambda i: (i, 0)),
        out_shape=jax.ShapeDtypeStruct((_T, _C), jnp.float32),
    )(qh, yh, kv3, gfull, wp, bp)


def kernel(x, memory_db, W_attn, b_attn, W_proj, b_proj, gate_bias):
    x2 = x.reshape(_T, _C)
    mem_flat = memory_db.reshape(_M, 2 * _C)
    qh, kth, vh, qt = _qkv(x2, W_attn, b_attn)
    idx = _topk(qt, mem_flat)
    kvs = _gather_rows(mem_flat, idx.reshape(_K * _T))
    kv3 = kvs.reshape(_K, _T, 2 * _C)
    yh = _attn(qh, kth, vh)
    gfull = jnp.repeat(gate_bias.reshape(_H), _DH)
    out = _combine(qh, yh, kv3, gfull, W_proj, b_proj)
    return out.reshape(_B, _T, _C)
